# baseline (speedup 1.0000x reference)
import functools

import jax
import jax.numpy as jnp
from jax import lax
from jax.experimental import pallas as pl
from jax.experimental.pallas import tpu as pltpu
from jax.experimental.pallas import tpu_sc as plsc

EMB_DIM = 32
N_I = 16384
N_J = 200
VOCAB = 65536
NUM_CORES = 2
NUM_SUBCORES = 16
NBUF = 3
LANES = 16
GROUPS = N_I // LANES

_mesh = plsc.VectorSubcoreMesh(
    core_axis_name="c", subcore_axis_name="s",
    num_cores=NUM_CORES, num_subcores=NUM_SUBCORES)


@functools.partial(
    pl.kernel,
    out_type=jax.ShapeDtypeStruct((N_J, EMB_DIM, N_I), jnp.float32),
    mesh=_mesh,
    scratch_types=[
        pltpu.VMEM((VOCAB,), jnp.float32),
        pltpu.VMEM((N_I,), jnp.float32),
        pltpu.VMEM((N_I,), jnp.float32),
        pltpu.VMEM((N_I,), jnp.float32),
        pltpu.SemaphoreType.DMA,
        pltpu.SemaphoreType.DMA,
    ],
    compiler_params=pltpu.CompilerParams(needs_layout_passes=False),
)
def _qc_embed(ftr_hbm, ttr_hbm, out_hbm, tbl_v, buf0_v, buf1_v, buf2_v,
              isem, wsem):
    bufs = [buf0_v, buf1_v, buf2_v]
    d = lax.axis_index("s") * NUM_CORES + lax.axis_index("c")

    pltpu.sync_copy(ttr_hbm.at[d], tbl_v)

    pltpu.async_copy(ftr_hbm.at[0, pl.ds(0, N_I)], bufs[0], isem)

    def row(jj, carry):
        for b in range(NBUF):
            @pl.when(lax.rem(jj, NBUF) == b)
            def _():
                @pl.when(jj >= 2)
                def _():
                    pltpu.make_async_copy(
                        bufs[(b + 1) % NBUF],
                        out_hbm.at[jj, d, pl.ds(0, N_I)], wsem,
                    ).wait()

                pltpu.make_async_copy(
                    ftr_hbm.at[jj, pl.ds(0, N_I)], bufs[b], isem).wait()

                @pl.when(jj + 1 < N_J)
                def _():
                    pltpu.async_copy(
                        ftr_hbm.at[jj + 1, pl.ds(0, N_I)],
                        bufs[(b + 1) % NBUF], isem)

                @plsc.parallel_loop(0, GROUPS, unroll=16)
                def _(g):
                    sl = pl.ds(g * LANES, LANES)
                    ids = plsc.bitcast(bufs[b][sl], jnp.int32) & jnp.int32(0xFFFF)
                    bufs[b][sl] = plsc.load_gather(tbl_v, [ids])

                pltpu.async_copy(bufs[b], out_hbm.at[jj, d, pl.ds(0, N_I)], wsem)
        return carry

    lax.fori_loop(0, N_J, row, 0)
    for _ in range(2):
        pltpu.make_async_copy(bufs[0], out_hbm.at[0, d, pl.ds(0, N_I)], wsem).wait()


def kernel(qc_flags, emb_table):
    flags_f32 = lax.bitcast_convert_type(qc_flags.T, jnp.float32)
    out3 = _qc_embed(flags_f32, emb_table.T)
    return out3.transpose(2, 0, 1)
